# baseline (speedup 1.0000x reference)
import functools

import jax
import jax.numpy as jnp
from jax import lax
from jax.experimental import pallas as pl
from jax.experimental.pallas import tpu as pltpu
from jax.experimental.pallas import tpu_sc as plsc

D = 128
NC = 2
NS = 16
NW = NC * NS
L = 16
CHUNK = 128
NBUF = 2
ID = 4
N_PAD = 10240
ROWS_PER_TILE = N_PAD // NS
STEPS_OUT = ROWS_PER_TILE // CHUNK


def _sc_aggregate(x, src2, dst2, n_chunks):
    mesh = plsc.VectorSubcoreMesh(core_axis_name="c", subcore_axis_name="s")

    @functools.partial(
        pl.kernel,
        out_type=(
            jax.ShapeDtypeStruct((NC * N_PAD, D), jnp.float32),
            jax.ShapeDtypeStruct((NW, N_PAD), jnp.float32),
        ),
        mesh=mesh,
        scratch_types=[
            pltpu.VMEM((ID, CHUNK), jnp.int32),
            pltpu.VMEM((ID, CHUNK), jnp.int32),
            pltpu.VMEM((NBUF, CHUNK, D), jnp.float32),
            pltpu.VMEM((N_PAD,), jnp.float32),
            pltpu.VMEM_SHARED((N_PAD, D), jnp.float32),
            pltpu.SemaphoreType.DMA((ID,)),
            pltpu.SemaphoreType.DMA((NBUF,)),
        ],
        compiler_params=pltpu.CompilerParams(needs_layout_passes=False),
    )
    def agg(x_hbm, src_hbm, dst_hbm, psum_hbm, hist_hbm,
            src_r, dst_r, rows2, hist_v, acc_sh, isem, gsem):
        cid = lax.axis_index("c")
        sid = lax.axis_index("s")
        wid = cid * NS + sid
        rows_v = rows2.at[0]
        irow = wid * n_chunks

        def zrow(i, _):
            def zcol(j, _):
                rows_v[i, pl.ds(j * L, L)] = jnp.zeros((L,), jnp.float32)
                return 0
            lax.fori_loop(0, D // L, zcol, 0)
            return 0
        lax.fori_loop(0, CHUNK, zrow, 0)

        def zhist(i, _):
            hist_v[pl.ds(i * L, L)] = jnp.zeros((L,), jnp.float32)
            return 0
        lax.fori_loop(0, N_PAD // L, zhist, 0)

        base = sid * ROWS_PER_TILE
        def zacc(t, _):
            pltpu.sync_copy(rows_v, acc_sh.at[pl.ds(base + t * CHUNK, CHUNK)])
            return 0
        lax.fori_loop(0, STEPS_OUT, zacc, 0)

        ones16 = jnp.ones((L,), jnp.float32)

        for k in range(ID):
            pltpu.async_copy(src_hbm.at[irow + k], src_r.at[k], isem.at[k])
            pltpu.async_copy(dst_hbm.at[irow + k], dst_r.at[k], isem.at[k])
        for b in range(NBUF):
            pltpu.make_async_copy(src_hbm.at[0], src_r.at[b], isem.at[b]).wait()
            pltpu.make_async_copy(dst_hbm.at[0], dst_r.at[b], isem.at[b]).wait()
            pltpu.async_copy(x_hbm.at[src_r.at[b]], rows2.at[b], gsem.at[b])

        plsc.subcore_barrier()

        def pipe_body(t, _):
            j = t * ID
            for b in range(ID):
                cur = j + b
                rb = b % NBUF
                def cnt(i, _):
                    idx16 = dst_r[b, pl.ds(i * L, L)]
                    plsc.addupdate_scatter(hist_v, [idx16], ones16)
                    return 0
                lax.fori_loop(0, CHUNK // L, cnt, 0)
                pltpu.make_async_copy(
                    x_hbm.at[src_r.at[b]], rows2.at[rb], gsem.at[rb]).wait()
                pltpu.sync_copy(rows2.at[rb], acc_sh.at[dst_r.at[b]], add=True)
                nxt_i = cur + ID
                @pl.when(nxt_i < n_chunks)
                def _():
                    pltpu.async_copy(
                        src_hbm.at[irow + nxt_i], src_r.at[b], isem.at[b])
                    pltpu.async_copy(
                        dst_hbm.at[irow + nxt_i], dst_r.at[b], isem.at[b])
                nxt_g = cur + NBUF
                kb = (b + NBUF) % ID
                @pl.when(nxt_g < n_chunks)
                def _():
                    pltpu.make_async_copy(
                        src_hbm.at[0], src_r.at[kb], isem.at[kb]).wait()
                    pltpu.make_async_copy(
                        dst_hbm.at[0], dst_r.at[kb], isem.at[kb]).wait()
                    pltpu.async_copy(
                        x_hbm.at[src_r.at[kb]], rows2.at[rb], gsem.at[rb])
            return 0
        lax.fori_loop(0, n_chunks // ID, pipe_body, 0)

        plsc.subcore_barrier()

        out_base = cid * N_PAD + base
        def wout(t, _):
            pltpu.sync_copy(acc_sh.at[pl.ds(base + t * CHUNK, CHUNK)], rows_v)
            pltpu.sync_copy(rows_v, psum_hbm.at[pl.ds(out_base + t * CHUNK, CHUNK)])
            return 0
        lax.fori_loop(0, STEPS_OUT, wout, 0)
        pltpu.sync_copy(hist_v, hist_hbm.at[wid])

    return agg(x, src2, dst2)


def _tc_combine(x_pad, psum, pcnt, W_self, W_neigh, b_self, b_neigh):
    blk = 1024
    grid = (N_PAD // blk,)

    def body(x_ref, ps_ref, pc_ref, ws_ref, wn_ref, bs_ref, bn_ref, o_ref):
        s = ps_ref[0] + ps_ref[1]
        cnt = jnp.sum(pc_ref[:], axis=0)[:, None]
        mean = s / jnp.maximum(cnt, 1.0)
        dn = (((1,), (1,)), ((), ()))
        o_ref[:] = (
            lax.dot_general(x_ref[:], ws_ref[:], dn,
                            preferred_element_type=jnp.float32)
            + lax.dot_general(mean, wn_ref[:], dn,
                              preferred_element_type=jnp.float32)
            + bs_ref[:] + bn_ref[:]
        )

    return pl.pallas_call(
        body,
        grid=grid,
        in_specs=[
            pl.BlockSpec((blk, D), lambda i: (i, 0)),
            pl.BlockSpec((NC, blk, D), lambda i: (0, i, 0)),
            pl.BlockSpec((NW, blk), lambda i: (0, i)),
            pl.BlockSpec((D, D), lambda i: (0, 0)),
            pl.BlockSpec((D, D), lambda i: (0, 0)),
            pl.BlockSpec((1, D), lambda i: (0, 0)),
            pl.BlockSpec((1, D), lambda i: (0, 0)),
        ],
        out_specs=pl.BlockSpec((blk, D), lambda i: (i, 0)),
        out_shape=jax.ShapeDtypeStruct((N_PAD, D), jnp.float32),
    )(x_pad, psum, pcnt, W_self, W_neigh,
      b_self.reshape(1, D), b_neigh.reshape(1, D))


def kernel(x, edge_index, W_self, b_self, W_neigh, b_neigh):
    n = x.shape[0]
    src = edge_index[0].astype(jnp.int32)
    dst = edge_index[1].astype(jnp.int32)
    e = src.shape[0]
    n_chunks = -(-e // (NW * CHUNK))
    n_chunks = max(-(-n_chunks // ID) * ID, ID)
    pad = NW * CHUNK * n_chunks - e
    src_p = jnp.concatenate([src, jnp.zeros((pad,), jnp.int32)])
    dst_p = jnp.concatenate([dst, jnp.full((pad,), N_PAD - 1, jnp.int32)])
    src3 = src_p.reshape(NW * n_chunks, CHUNK)
    dst3 = dst_p.reshape(NW * n_chunks, CHUNK)

    psum, pcnt = _sc_aggregate(x, src3, dst3, n_chunks)

    x_pad = jnp.pad(x, ((0, N_PAD - n), (0, 0)))
    out = _tc_combine(
        x_pad,
        psum.reshape(NC, N_PAD, D),
        pcnt,
        W_self, W_neigh, b_self, b_neigh,
    )
    return out[:n]


# speedup vs baseline: 1.3330x; 1.3330x over previous
import functools

import jax
import jax.numpy as jnp
from jax import lax
from jax.experimental import pallas as pl
from jax.experimental.pallas import tpu as pltpu
from jax.experimental.pallas import tpu_sc as plsc

D = 128
NC = 2
NS = 16
NW = NC * NS
L = 16
CHUNK = 128
NBUF = 1
N_PAD = 10240
ROWS_PER_TILE = N_PAD // NS
STEPS_OUT = ROWS_PER_TILE // CHUNK


def _sc_aggregate(x, src2, dst2, n_chunks):
    mesh = plsc.VectorSubcoreMesh(core_axis_name="c", subcore_axis_name="s")

    @functools.partial(
        pl.kernel,
        out_type=(
            jax.ShapeDtypeStruct((NC * N_PAD, D), jnp.float32),
            jax.ShapeDtypeStruct((NW, N_PAD), jnp.float32),
        ),
        mesh=mesh,
        scratch_types=[
            pltpu.VMEM((n_chunks, CHUNK), jnp.int32),
            pltpu.VMEM((n_chunks, CHUNK), jnp.int32),
            pltpu.VMEM((NBUF, CHUNK, D), jnp.float32),
            pltpu.VMEM((N_PAD,), jnp.float32),
            pltpu.VMEM_SHARED((N_PAD, D), jnp.float32),
            pltpu.SemaphoreType.DMA((NBUF,)),
        ],
        compiler_params=pltpu.CompilerParams(needs_layout_passes=False),
    )
    def agg(x_hbm, src_hbm, dst_hbm, psum_hbm, hist_hbm,
            src_v, dst_v, rows2, hist_v, acc_sh, gsem):
        cid = lax.axis_index("c")
        sid = lax.axis_index("s")
        wid = cid * NS + sid
        rows_v = rows2.at[0]

        def zrow(i, _):
            def zcol(j, _):
                rows_v[i, pl.ds(j * L, L)] = jnp.zeros((L,), jnp.float32)
                return 0
            lax.fori_loop(0, D // L, zcol, 0)
            return 0
        lax.fori_loop(0, CHUNK, zrow, 0)

        def zhist(i, _):
            hist_v[pl.ds(i * L, L)] = jnp.zeros((L,), jnp.float32)
            return 0
        lax.fori_loop(0, N_PAD // L, zhist, 0)

        base = sid * ROWS_PER_TILE
        def zacc(t, _):
            pltpu.sync_copy(rows_v, acc_sh.at[pl.ds(base + t * CHUNK, CHUNK)])
            return 0
        lax.fori_loop(0, STEPS_OUT, zacc, 0)

        pltpu.sync_copy(src_hbm.at[wid], src_v)
        pltpu.sync_copy(dst_hbm.at[wid], dst_v)

        ones16 = jnp.ones((L,), jnp.float32)

        for b in range(NBUF):
            pltpu.async_copy(x_hbm.at[src_v.at[b]], rows2.at[b], gsem.at[b])

        plsc.subcore_barrier()

        def pipe_body(t, _):
            j = t * NBUF
            for b in range(NBUF):
                cur = j + b
                def cnt(i, _):
                    idx16 = dst_v[cur, pl.ds(i * L, L)]
                    plsc.addupdate_scatter(hist_v, [idx16], ones16)
                    return 0
                lax.fori_loop(0, CHUNK // L, cnt, 0)
                pltpu.make_async_copy(
                    x_hbm.at[src_v.at[cur]], rows2.at[b], gsem.at[b]).wait()
                pltpu.sync_copy(rows2.at[b], acc_sh.at[dst_v.at[cur]], add=True)
                nxt = cur + NBUF
                @pl.when(nxt < n_chunks)
                def _():
                    pltpu.async_copy(
                        x_hbm.at[src_v.at[nxt]], rows2.at[b], gsem.at[b])
            return 0
        lax.fori_loop(0, n_chunks // NBUF, pipe_body, 0)

        plsc.subcore_barrier()

        out_base = cid * N_PAD + base
        def wout(t, _):
            pltpu.sync_copy(acc_sh.at[pl.ds(base + t * CHUNK, CHUNK)], rows_v)
            pltpu.sync_copy(rows_v, psum_hbm.at[pl.ds(out_base + t * CHUNK, CHUNK)])
            return 0
        lax.fori_loop(0, STEPS_OUT, wout, 0)
        pltpu.sync_copy(hist_v, hist_hbm.at[wid])

    return agg(x, src2, dst2)


def _tc_combine(x_pad, psum, pcnt, W_self, W_neigh, b_self, b_neigh):
    blk = 1024
    grid = (N_PAD // blk,)

    def body(x_ref, ps_ref, pc_ref, ws_ref, wn_ref, bs_ref, bn_ref, o_ref):
        s = ps_ref[0] + ps_ref[1]
        cnt = jnp.sum(pc_ref[:], axis=0)[:, None]
        mean = s / jnp.maximum(cnt, 1.0)
        dn = (((1,), (1,)), ((), ()))
        o_ref[:] = (
            lax.dot_general(x_ref[:], ws_ref[:], dn,
                            preferred_element_type=jnp.float32)
            + lax.dot_general(mean, wn_ref[:], dn,
                              preferred_element_type=jnp.float32)
            + bs_ref[:] + bn_ref[:]
        )

    return pl.pallas_call(
        body,
        grid=grid,
        in_specs=[
            pl.BlockSpec((blk, D), lambda i: (i, 0)),
            pl.BlockSpec((NC, blk, D), lambda i: (0, i, 0)),
            pl.BlockSpec((NW, blk), lambda i: (0, i)),
            pl.BlockSpec((D, D), lambda i: (0, 0)),
            pl.BlockSpec((D, D), lambda i: (0, 0)),
            pl.BlockSpec((1, D), lambda i: (0, 0)),
            pl.BlockSpec((1, D), lambda i: (0, 0)),
        ],
        out_specs=pl.BlockSpec((blk, D), lambda i: (i, 0)),
        out_shape=jax.ShapeDtypeStruct((N_PAD, D), jnp.float32),
    )(x_pad, psum, pcnt, W_self, W_neigh,
      b_self.reshape(1, D), b_neigh.reshape(1, D))


def kernel(x, edge_index, W_self, b_self, W_neigh, b_neigh):
    n = x.shape[0]
    src = edge_index[0].astype(jnp.int32)
    dst = edge_index[1].astype(jnp.int32)
    e = src.shape[0]
    n_chunks = -(-e // (NW * CHUNK))
    n_chunks = max(-(-n_chunks // NBUF) * NBUF, NBUF)
    pad = NW * CHUNK * n_chunks - e
    src_p = jnp.concatenate([src, jnp.zeros((pad,), jnp.int32)])
    dst_p = jnp.concatenate([dst, jnp.full((pad,), N_PAD - 1, jnp.int32)])
    src3 = src_p.reshape(NW, n_chunks, CHUNK)
    dst3 = dst_p.reshape(NW, n_chunks, CHUNK)

    psum, pcnt = _sc_aggregate(x, src3, dst3, n_chunks)

    x_pad = jnp.pad(x, ((0, N_PAD - n), (0, 0)))
    out = _tc_combine(
        x_pad,
        psum.reshape(NC, N_PAD, D),
        pcnt,
        W_self, W_neigh, b_self, b_neigh,
    )
    return out[:n]
